# baseline (speedup 1.0000x reference)
import functools

import jax
import jax.numpy as jnp
from jax import lax
from jax.experimental import pallas as pl
from jax.experimental.pallas import tpu as pltpu
from jax.experimental.pallas import tpu_sc as plsc

ROWS = 64
D = 64
NROW = 1000000
BASE = 7
STRIDE = 15625
L = 16
NWORK = 8
KPW = ROWS // NWORK
NIDX = KPW * D
IROWS = NIDX // 128


def kernel(a):
    info = plsc.get_sparse_core_info()
    nc = info.num_cores

    mesh = plsc.VectorSubcoreMesh(core_axis_name="c", subcore_axis_name="s")

    @functools.partial(
        pl.kernel,
        mesh=mesh,
        out_type=jax.ShapeDtypeStruct((ROWS, D), jnp.float32),
        scratch_types=[
            pltpu.VMEM((IROWS, 128), jnp.int32),
            pltpu.VMEM((IROWS, 128), jnp.float32),
            pltpu.VMEM((KPW, D), jnp.float32),
            pltpu.SemaphoreType.DMA,
        ],
        compiler_params=pltpu.CompilerParams(needs_layout_passes=False),
    )
    def sc_gather(flat_hbm, out_hbm, idx_v, gbuf, obuf, sem):
        wid = lax.axis_index("s") * nc + lax.axis_index("c")

        @pl.when(wid < NWORK)
        def _():
            row0 = wid * KPW
            for j in range(IROWS):
                for t in range(128 // L):
                    n0 = j * 128 + t * L
                    kk = n0 // D
                    c0 = n0 % D
                    idx_v[j, pl.ds(t * L, L)] = (
                        lax.iota(jnp.int32, L) * NROW
                        + (c0 * NROW + BASE)
                        + (row0 + kk) * STRIDE
                    )
            copies = [
                pltpu.async_copy(flat_hbm.at[idx_v.at[j]], gbuf.at[j], sem)
                for j in range(IROWS)
            ]
            for cp in copies:
                cp.wait()
            for i in range(KPW):
                for g in range(D // L):
                    n0 = i * D + g * L
                    obuf[i, pl.ds(g * L, L)] = (
                        gbuf[n0 // 128, pl.ds(n0 % 128, L)] * 2.0
                    )
            pltpu.sync_copy(obuf, out_hbm.at[pl.ds(row0, KPW)])

    return sc_gather(a.T.reshape(-1))


# speedup vs baseline: 217.6254x; 217.6254x over previous
import functools

import jax
import jax.numpy as jnp
from jax import lax
from jax.experimental import pallas as pl
from jax.experimental.pallas import tpu as pltpu
from jax.experimental.pallas import tpu_sc as plsc

ROWS = 64
D = 64
BASE = 7
STRIDE = 15625
L = 16
NWORK = 8
KPW = ROWS // NWORK


def kernel(a):
    info = plsc.get_sparse_core_info()
    nc = info.num_cores

    mesh = plsc.VectorSubcoreMesh(core_axis_name="c", subcore_axis_name="s")

    @functools.partial(
        pl.kernel,
        mesh=mesh,
        out_type=jax.ShapeDtypeStruct((ROWS, D), jnp.float32),
        scratch_types=[
            pltpu.VMEM((KPW, D, 128), jnp.float32),
            pltpu.VMEM((KPW, D), jnp.float32),
            pltpu.SemaphoreType.DMA((KPW,)),
        ],
        compiler_params=pltpu.CompilerParams(needs_layout_passes=False),
    )
    def sc_gather(att_hbm, out_hbm, blocks_v, rows_v, sem):
        wid = lax.axis_index("s") * nc + lax.axis_index("c")

        @pl.when(wid < NWORK)
        def _():
            base0 = BASE + wid * (KPW * STRIDE)
            copies = []
            for i in range(KPW):
                idx = base0 + i * STRIDE
                q0 = pl.multiple_of((idx // 128) * 128, 128)
                copies.append(
                    pltpu.async_copy(
                        att_hbm.at[:, pl.ds(q0, 128)],
                        blocks_v.at[i],
                        sem.at[i],
                    )
                )
            for i in range(KPW):
                copies[i].wait()
                idx = base0 + i * STRIDE
                col = jnp.full((L,), lax.rem(idx, 128), dtype=jnp.int32)
                for g in range(D // L):
                    row = lax.iota(jnp.int32, L) + g * L
                    vals = plsc.load_gather(blocks_v.at[i], [row, col])
                    rows_v[i, pl.ds(g * L, L)] = vals * 2.0
            pltpu.sync_copy(rows_v, out_hbm.at[pl.ds(wid * KPW, KPW)])

    return sc_gather(a.T)


# speedup vs baseline: 234.7895x; 1.0789x over previous
import functools

import jax
import jax.numpy as jnp
from jax import lax
from jax.experimental import pallas as pl
from jax.experimental.pallas import tpu as pltpu
from jax.experimental.pallas import tpu_sc as plsc

ROWS = 64
D = 64
BASE = 7
STRIDE = 15625
L = 16
NWORK = 8
KPW = ROWS // NWORK


def kernel(a):
    info = plsc.get_sparse_core_info()
    nc = info.num_cores

    mesh = plsc.VectorSubcoreMesh(
        core_axis_name="c", subcore_axis_name="s", num_cores=1
    )

    @functools.partial(
        pl.kernel,
        mesh=mesh,
        out_type=jax.ShapeDtypeStruct((ROWS, D), jnp.float32),
        scratch_types=[
            pltpu.VMEM((KPW, D, 128), jnp.float32),
            pltpu.VMEM((KPW, D), jnp.float32),
            pltpu.SemaphoreType.DMA,
        ],
        compiler_params=pltpu.CompilerParams(needs_layout_passes=False),
    )
    def sc_gather(att_hbm, out_hbm, blocks_v, rows_v, sem):
        wid = lax.axis_index("s") * nc + lax.axis_index("c")

        @pl.when(wid < NWORK)
        def _():
            base0 = BASE + wid * (KPW * STRIDE)
            copies = []
            for i in range(KPW):
                idx = base0 + i * STRIDE
                q0 = pl.multiple_of((idx // 128) * 128, 128)
                copies.append(
                    pltpu.async_copy(
                        att_hbm.at[:, pl.ds(q0, 128)], blocks_v.at[i], sem
                    )
                )
            for cp in copies:
                cp.wait()
            for i in range(KPW):
                idx = base0 + i * STRIDE
                col = jnp.full((L,), lax.rem(idx, 128), dtype=jnp.int32)
                for g in range(D // L):
                    row = lax.iota(jnp.int32, L) + g * L
                    vals = plsc.load_gather(blocks_v.at[i], [row, col])
                    rows_v[i, pl.ds(g * L, L)] = vals * 2.0
            pltpu.sync_copy(rows_v, out_hbm.at[pl.ds(wid * KPW, KPW)])

    return sc_gather(a.T)
